# baseline (speedup 1.0000x reference)
import functools

import jax
import jax.numpy as jnp
from jax import lax
from jax.experimental import pallas as pl
from jax.experimental.pallas import tpu as pltpu
from jax.experimental.pallas import tpu_sc as plsc

_N = 10000
_E = 320000
_D = 128
_H = 128
_ODIM = 121
_HOPS = 3

_NP = 10240
_NW = 32
_NT = 16
_CK = 64
_TOTCH = 5120
_BCH = 32
_CH0 = 96
_CH1 = 224
_EP = _TOTCH * _CK
_SLC = _NP // _NT
_BLK = 256
_GRID = _NP // _BLK


def _pre_body(x_ref, wi_ref, bi_ref, wg_ref, asrc_ref, adst_ref,
              x0_ref, wh_ref, ss_ref, sd_ref, gmax_ref):
    x0 = jnp.maximum(
        jnp.dot(x_ref[...], wi_ref[...], preferred_element_type=jnp.float32)
        + bi_ref[...], 0.0)
    x0_ref[...] = x0
    wh = jnp.dot(x0, wg_ref[...], preferred_element_type=jnp.float32)
    wh_ref[...] = wh
    ss = jnp.dot(wh, asrc_ref[...], preferred_element_type=jnp.float32)
    ss_ref[...] = ss
    sd_ref[...] = jnp.dot(wh, adst_ref[...], preferred_element_type=jnp.float32)
    m = jnp.full((1, 1), jnp.max(ss), jnp.float32)

    @pl.when(pl.program_id(0) == 0)
    def _():
        gmax_ref[...] = m

    @pl.when(pl.program_id(0) > 0)
    def _():
        gmax_ref[...] = jnp.maximum(gmax_ref[...], m)


def _mid_body(agg_ref, d0_ref, d1_ref, wg_ref, asrc_ref, adst_ref,
              h_ref, wh_ref, ss_ref, sd_ref, gmax_ref):
    invd = 1.0 / (d0_ref[...] + d1_ref[...] + 1e-16)
    h = jnp.tanh((agg_ref[0] + agg_ref[1]) * invd)
    h_ref[...] = h
    wh = jnp.dot(h, wg_ref[...], preferred_element_type=jnp.float32)
    wh_ref[...] = wh
    ss = jnp.dot(wh, asrc_ref[...], preferred_element_type=jnp.float32)
    ss_ref[...] = ss
    sd_ref[...] = jnp.dot(wh, adst_ref[...], preferred_element_type=jnp.float32)
    m = jnp.full((1, 1), jnp.max(ss), jnp.float32)

    @pl.when(pl.program_id(0) == 0)
    def _():
        gmax_ref[...] = m

    @pl.when(pl.program_id(0) > 0)
    def _():
        gmax_ref[...] = jnp.maximum(gmax_ref[...], m)


def _fin_body(agg_ref, d0_ref, d1_ref, x0_ref, h1_ref, h2_ref,
              wih_ref, whh_ref, bl_ref, wo_ref, bo_ref, out_ref):
    invd = 1.0 / (d0_ref[...] + d1_ref[...] + 1e-16)
    h3 = jnp.tanh((agg_ref[0] + agg_ref[1]) * invd)
    x0 = x0_ref[...]
    hs = jnp.zeros((_BLK, _H), jnp.float32)
    cs = jnp.zeros((_BLK, _H), jnp.float32)
    for ht in (h1_ref[...], h2_ref[...], h3):
        inp = ht + x0
        gates = (jnp.dot(inp, wih_ref[...], preferred_element_type=jnp.float32)
                 + jnp.dot(hs, whh_ref[...], preferred_element_type=jnp.float32)
                 + bl_ref[...])
        ig = jax.nn.sigmoid(gates[:, 0:_H])
        fg = jax.nn.sigmoid(gates[:, _H:2 * _H])
        gg = jnp.tanh(gates[:, 2 * _H:3 * _H])
        og = jax.nn.sigmoid(gates[:, 3 * _H:4 * _H])
        cs = fg * cs + ig * gg
        hs = og * jnp.tanh(cs)
    out_ref[...] = (jnp.dot(hs, wo_ref[...], preferred_element_type=jnp.float32)
                    + bo_ref[...])


_f32 = jnp.float32


def _vec_specs():
    full = lambda shape: pl.BlockSpec(shape, lambda i: (0,) * len(shape))
    row = pl.BlockSpec((_BLK, _D), lambda i: (i, 0))
    col = pl.BlockSpec((_BLK, 1), lambda i: (i, 0))
    return full, row, col


def _tc_pre(xp, W_init, b_init, Wg, asrc, adst):
    full, row, col = _vec_specs()
    return pl.pallas_call(
        _pre_body,
        grid=(_GRID,),
        in_specs=[row, full((_D, _H)), full((1, _H)), full((_H, _H)),
                  full((_H, 1)), full((_H, 1))],
        out_specs=[row, row, col, col, full((1, 1))],
        out_shape=[jax.ShapeDtypeStruct((_NP, _H), _f32),
                   jax.ShapeDtypeStruct((_NP, _H), _f32),
                   jax.ShapeDtypeStruct((_NP, 1), _f32),
                   jax.ShapeDtypeStruct((_NP, 1), _f32),
                   jax.ShapeDtypeStruct((1, 1), _f32)],
    )(xp, W_init, b_init, Wg, asrc, adst)


def _tc_mid(aggp, d0, d1, Wg, asrc, adst):
    full, row, col = _vec_specs()
    agg_spec = pl.BlockSpec((2, _BLK, _H), lambda i: (0, i, 0))
    return pl.pallas_call(
        _mid_body,
        grid=(_GRID,),
        in_specs=[agg_spec, col, col, full((_H, _H)),
                  full((_H, 1)), full((_H, 1))],
        out_specs=[row, row, col, col, full((1, 1))],
        out_shape=[jax.ShapeDtypeStruct((_NP, _H), _f32),
                   jax.ShapeDtypeStruct((_NP, _H), _f32),
                   jax.ShapeDtypeStruct((_NP, 1), _f32),
                   jax.ShapeDtypeStruct((_NP, 1), _f32),
                   jax.ShapeDtypeStruct((1, 1), _f32)],
    )(aggp, d0, d1, Wg, asrc, adst)


def _tc_fin(aggp, d0, d1, x0p, h1, h2, W_ih, W_hh, bl, Wo, bo):
    full, row, col = _vec_specs()
    agg_spec = pl.BlockSpec((2, _BLK, _H), lambda i: (0, i, 0))
    return pl.pallas_call(
        _fin_body,
        grid=(_GRID,),
        in_specs=[agg_spec, col, col, row, row, row,
                  full((_H, 4 * _H)), full((_H, 4 * _H)), full((1, 4 * _H)),
                  full((_H, _H)), full((1, _H))],
        out_specs=[row],
        out_shape=[jax.ShapeDtypeStruct((_NP, _H), _f32)],
    )(aggp, d0, d1, x0p, h1, h2, W_ih, W_hh, bl, Wo, bo)


def _sc_body(ssrc_hbm, sdst_hbm, gmax_hbm, src_hbm, dst_hbm, wh_hbm,
             agg_out, den_out,
             ssrc_t, sdst_t, srcb, dstb, exj, rows, gmaxv,
             agg_sh, den_sh, sem_g0, sem_g1):
    c = lax.axis_index("c")
    s = lax.axis_index("s")
    base = s * _SLC

    pltpu.sync_copy(ssrc_hbm, ssrc_t)
    pltpu.sync_copy(sdst_hbm, sdst_t)
    pltpu.sync_copy(gmax_hbm, gmaxv)

    zv = jnp.zeros((16,), jnp.float32)

    def _zero_rows(i, carry):
        for q in range(8):
            rows[0, i, pl.ds(q * 16, 16)] = zv
        return carry

    lax.fori_loop(0, _CK, _zero_rows, 0)
    for q in range(4):
        exj[pl.ds(q * 16, 16)] = zv

    for k in range(_SLC // _CK):
        pltpu.sync_copy(rows.at[0], agg_sh.at[pl.ds(base + k * _CK, _CK)])
        pltpu.sync_copy(exj, den_sh.at[pl.ds(base + k * _CK, _CK)])
    plsc.subcore_barrier()

    gm = gmaxv[...]
    sems = (sem_g0, sem_g1)

    start_ch = jnp.where(c == 0, s * _CH0, 16 * _CH0 + s * _CH1)
    nblocks = jnp.where(c == 0, _CH0 // _BCH, _CH1 // _BCH)

    def _issue_gather(i, q):
        pltpu.async_copy(wh_hbm.at[srcb.at[i]], rows.at[q], sems[q])

    def _process(i, p):
        for m in range(4):
            sv = srcb[i, pl.ds(m * 16, 16)]
            dv = dstb[i, pl.ds(m * 16, 16)]
            s1 = plsc.load_gather(ssrc_t, [sv])
            s2 = plsc.load_gather(sdst_t, [dv])
            z = s1 + s2
            e = jnp.where(z >= 0.0, z, 0.2 * z)
            ex = jnp.exp(e - jnp.maximum(s2 + gm, 0.0))
            exj[pl.ds(m * 16, 16)] = ex
        pltpu.sync_copy(exj, den_sh.at[dstb.at[i]], add=True)
        pltpu.make_async_copy(wh_hbm.at[srcb.at[i]], rows.at[p], sems[p]).wait()

        def _scale(g, carry):
            exv = exj[pl.ds(g * 16, 16)]
            for r16 in range(16):
                r = g * 16 + r16
                av = jnp.full((16,), exv[r16], jnp.float32)
                for q in range(8):
                    rows[p, r, pl.ds(q * 16, 16)] = (
                        rows[p, r, pl.ds(q * 16, 16)] * av)
            return carry

        lax.fori_loop(0, _CK // 16, _scale, 0)
        pltpu.sync_copy(rows.at[p], agg_sh.at[dstb.at[i]], add=True)

    def _block(b, carry):
        pltpu.sync_copy(src_hbm.at[pl.ds(start_ch + b * _BCH, _BCH)], srcb)
        pltpu.sync_copy(dst_hbm.at[pl.ds(start_ch + b * _BCH, _BCH)], dstb)
        _issue_gather(0, 0)

        def _pair(i2, carry2):
            i0 = 2 * i2
            _issue_gather(i0 + 1, 1)
            _process(i0, 0)
            _issue_gather(i0 + 2, 0)
            _process(i0 + 1, 1)
            return carry2

        lax.fori_loop(0, _BCH // 2 - 1, _pair, 0)
        _issue_gather(_BCH - 1, 1)
        _process(_BCH - 2, 0)
        _process(_BCH - 1, 1)
        return carry

    lax.fori_loop(0, nblocks, _block, 0)
    plsc.subcore_barrier()
    pltpu.sync_copy(agg_sh.at[pl.ds(base, _SLC)],
                    agg_out.at[c, pl.ds(base, _SLC)])
    pltpu.sync_copy(den_sh.at[pl.ds(base, _SLC)],
                    den_out.at[c, pl.ds(base, _SLC)])


def _sc_hop(ssrc, sdst, gmax16, srcp, dstp, wh):
    mesh = plsc.VectorSubcoreMesh(core_axis_name="c", subcore_axis_name="s")
    fn = pl.kernel(
        _sc_body,
        mesh=mesh,
        out_type=[jax.ShapeDtypeStruct((2, _NP, _H), _f32),
                  jax.ShapeDtypeStruct((2, _NP), _f32)],
        scratch_types=[
            pltpu.VMEM((_NP,), _f32),
            pltpu.VMEM((_NP,), _f32),
            pltpu.VMEM((_BCH, _CK), jnp.int32),
            pltpu.VMEM((_BCH, _CK), jnp.int32),
            pltpu.VMEM((_CK,), _f32),
            pltpu.VMEM((2, _CK, _H), _f32),
            pltpu.VMEM((16,), _f32),
            pltpu.VMEM_SHARED((_NP, _H), _f32),
            pltpu.VMEM_SHARED((_NP,), _f32),
            pltpu.SemaphoreType.DMA,
            pltpu.SemaphoreType.DMA,
        ],
        compiler_params=pltpu.CompilerParams(needs_layout_passes=False),
    )
    return fn(ssrc, sdst, gmax16, srcp, dstp, wh)


def kernel(x, edge_index, W_init, b_init, W_gat, a_src, a_dst,
           W_ih, W_hh, b_lstm, W_out, b_out):
    src = edge_index[0]
    dst = edge_index[1]
    srcp = jnp.concatenate(
        [src, jnp.zeros((_EP - _E,), jnp.int32)]).reshape(_TOTCH, _CK)
    dstp = jnp.concatenate(
        [dst, jnp.full((_EP - _E,), _NP - 1, jnp.int32)]).reshape(_TOTCH, _CK)
    xp = jnp.pad(x, ((0, _NP - _N), (0, 0)))

    bi = b_init.reshape(1, _H)
    bl = b_lstm.reshape(1, 4 * _H)
    Wo = jnp.pad(W_out, ((0, 0), (0, _H - _ODIM)))
    bo = jnp.pad(b_out, ((0, _H - _ODIM),)).reshape(1, _H)

    x0p, wh, ss, sd, gmax = _tc_pre(
        xp, W_init, bi, W_gat[0], a_src[0].reshape(_H, 1), a_dst[0].reshape(_H, 1))

    hs_list = []
    for hop in range(_HOPS):
        gmax16 = jnp.full((16,), gmax[0, 0], _f32)
        aggp, denp = _sc_hop(ss.reshape(_NP), sd.reshape(_NP), gmax16,
                             srcp, dstp, wh)
        d0 = denp[0].reshape(_NP, 1)
        d1 = denp[1].reshape(_NP, 1)
        if hop < _HOPS - 1:
            h, wh, ss, sd, gmax = _tc_mid(
                aggp, d0, d1, W_gat[hop + 1],
                a_src[hop + 1].reshape(_H, 1), a_dst[hop + 1].reshape(_H, 1))
            hs_list.append(h)
        else:
            (out,) = _tc_fin(aggp, d0, d1, x0p, hs_list[0], hs_list[1],
                             W_ih, W_hh, bl, Wo, bo)
    return out[:_N, :_ODIM]


# speedup vs baseline: 1.1550x; 1.1550x over previous
import functools

import jax
import jax.numpy as jnp
from jax import lax
from jax.experimental import pallas as pl
from jax.experimental.pallas import tpu as pltpu
from jax.experimental.pallas import tpu_sc as plsc

_N = 10000
_E = 320000
_D = 128
_H = 128
_ODIM = 121
_HOPS = 3

_NP = 10240
_NW = 32
_NT = 16
_CK = 64
_TOTCH = 5120
_BCH = 32
_CH0 = 224
_CH1 = 96
_EP = _TOTCH * _CK
_SLC = _NP // _NT
_BLK = 256
_GRID = _NP // _BLK


def _pre_body(x_ref, wi_ref, bi_ref, wg_ref, asrc_ref, adst_ref,
              x0_ref, wh_ref, ss_ref, sd_ref, gmax_ref):
    x0 = jnp.maximum(
        jnp.dot(x_ref[...], wi_ref[...], preferred_element_type=jnp.float32)
        + bi_ref[...], 0.0)
    x0_ref[...] = x0
    wh = jnp.dot(x0, wg_ref[...], preferred_element_type=jnp.float32)
    wh_ref[...] = wh
    ss = jnp.dot(wh, asrc_ref[...], preferred_element_type=jnp.float32)
    ss_ref[...] = ss
    sd_ref[...] = jnp.dot(wh, adst_ref[...], preferred_element_type=jnp.float32)
    m = jnp.full((1, 1), jnp.max(ss), jnp.float32)

    @pl.when(pl.program_id(0) == 0)
    def _():
        gmax_ref[...] = m

    @pl.when(pl.program_id(0) > 0)
    def _():
        gmax_ref[...] = jnp.maximum(gmax_ref[...], m)


def _mid_body(agg_ref, d0_ref, d1_ref, wg_ref, asrc_ref, adst_ref,
              h_ref, wh_ref, ss_ref, sd_ref, gmax_ref):
    invd = 1.0 / (d0_ref[...] + d1_ref[...] + 1e-16)
    h = jnp.tanh((agg_ref[0] + agg_ref[1]) * invd)
    h_ref[...] = h
    wh = jnp.dot(h, wg_ref[...], preferred_element_type=jnp.float32)
    wh_ref[...] = wh
    ss = jnp.dot(wh, asrc_ref[...], preferred_element_type=jnp.float32)
    ss_ref[...] = ss
    sd_ref[...] = jnp.dot(wh, adst_ref[...], preferred_element_type=jnp.float32)
    m = jnp.full((1, 1), jnp.max(ss), jnp.float32)

    @pl.when(pl.program_id(0) == 0)
    def _():
        gmax_ref[...] = m

    @pl.when(pl.program_id(0) > 0)
    def _():
        gmax_ref[...] = jnp.maximum(gmax_ref[...], m)


def _fin_body(agg_ref, d0_ref, d1_ref, x0_ref, h1_ref, h2_ref,
              wih_ref, whh_ref, bl_ref, wo_ref, bo_ref, out_ref):
    invd = 1.0 / (d0_ref[...] + d1_ref[...] + 1e-16)
    h3 = jnp.tanh((agg_ref[0] + agg_ref[1]) * invd)
    x0 = x0_ref[...]
    hs = jnp.zeros((_BLK, _H), jnp.float32)
    cs = jnp.zeros((_BLK, _H), jnp.float32)
    for ht in (h1_ref[...], h2_ref[...], h3):
        inp = ht + x0
        gates = (jnp.dot(inp, wih_ref[...], preferred_element_type=jnp.float32)
                 + jnp.dot(hs, whh_ref[...], preferred_element_type=jnp.float32)
                 + bl_ref[...])
        ig = jax.nn.sigmoid(gates[:, 0:_H])
        fg = jax.nn.sigmoid(gates[:, _H:2 * _H])
        gg = jnp.tanh(gates[:, 2 * _H:3 * _H])
        og = jax.nn.sigmoid(gates[:, 3 * _H:4 * _H])
        cs = fg * cs + ig * gg
        hs = og * jnp.tanh(cs)
    out_ref[...] = (jnp.dot(hs, wo_ref[...], preferred_element_type=jnp.float32)
                    + bo_ref[...])


_f32 = jnp.float32


def _vec_specs():
    full = lambda shape: pl.BlockSpec(shape, lambda i: (0,) * len(shape))
    row = pl.BlockSpec((_BLK, _D), lambda i: (i, 0))
    col = pl.BlockSpec((_BLK, 1), lambda i: (i, 0))
    return full, row, col


def _tc_pre(xp, W_init, b_init, Wg, asrc, adst):
    full, row, col = _vec_specs()
    return pl.pallas_call(
        _pre_body,
        grid=(_GRID,),
        in_specs=[row, full((_D, _H)), full((1, _H)), full((_H, _H)),
                  full((_H, 1)), full((_H, 1))],
        out_specs=[row, row, col, col, full((1, 1))],
        out_shape=[jax.ShapeDtypeStruct((_NP, _H), _f32),
                   jax.ShapeDtypeStruct((_NP, _H), _f32),
                   jax.ShapeDtypeStruct((_NP, 1), _f32),
                   jax.ShapeDtypeStruct((_NP, 1), _f32),
                   jax.ShapeDtypeStruct((1, 1), _f32)],
    )(xp, W_init, b_init, Wg, asrc, adst)


def _tc_mid(aggp, d0, d1, Wg, asrc, adst):
    full, row, col = _vec_specs()
    agg_spec = pl.BlockSpec((2, _BLK, _H), lambda i: (0, i, 0))
    return pl.pallas_call(
        _mid_body,
        grid=(_GRID,),
        in_specs=[agg_spec, col, col, full((_H, _H)),
                  full((_H, 1)), full((_H, 1))],
        out_specs=[row, row, col, col, full((1, 1))],
        out_shape=[jax.ShapeDtypeStruct((_NP, _H), _f32),
                   jax.ShapeDtypeStruct((_NP, _H), _f32),
                   jax.ShapeDtypeStruct((_NP, 1), _f32),
                   jax.ShapeDtypeStruct((_NP, 1), _f32),
                   jax.ShapeDtypeStruct((1, 1), _f32)],
    )(aggp, d0, d1, Wg, asrc, adst)


def _tc_fin(aggp, d0, d1, x0p, h1, h2, W_ih, W_hh, bl, Wo, bo):
    full, row, col = _vec_specs()
    agg_spec = pl.BlockSpec((2, _BLK, _H), lambda i: (0, i, 0))
    return pl.pallas_call(
        _fin_body,
        grid=(_GRID,),
        in_specs=[agg_spec, col, col, row, row, row,
                  full((_H, 4 * _H)), full((_H, 4 * _H)), full((1, 4 * _H)),
                  full((_H, _H)), full((1, _H))],
        out_specs=[row],
        out_shape=[jax.ShapeDtypeStruct((_NP, _H), _f32)],
    )(aggp, d0, d1, x0p, h1, h2, W_ih, W_hh, bl, Wo, bo)


def _sc_body(ssrc_hbm, sdst_hbm, gmax_hbm, src_hbm, dst_hbm, wh_hbm,
             agg_out, den_out,
             ssrc_t, sdst_t, srcb, dstb, exj, rows, gmaxv,
             agg_sh, den_sh, sem_g0, sem_g1):
    c = lax.axis_index("c")
    s = lax.axis_index("s")
    base = s * _SLC

    pltpu.sync_copy(ssrc_hbm, ssrc_t)
    pltpu.sync_copy(sdst_hbm, sdst_t)
    pltpu.sync_copy(gmax_hbm, gmaxv)

    zv = jnp.zeros((16,), jnp.float32)

    def _zero_rows(i, carry):
        for q in range(8):
            rows[0, i, pl.ds(q * 16, 16)] = zv
        return carry

    lax.fori_loop(0, _CK, _zero_rows, 0)
    for q in range(4):
        exj[pl.ds(q * 16, 16)] = zv

    for k in range(_SLC // _CK):
        pltpu.sync_copy(rows.at[0], agg_sh.at[pl.ds(base + k * _CK, _CK)])
        pltpu.sync_copy(exj, den_sh.at[pl.ds(base + k * _CK, _CK)])
    plsc.subcore_barrier()

    gm = gmaxv[...]
    sems = (sem_g0, sem_g1)

    start_ch = jnp.where(c == 0, s * _CH0, 16 * _CH0 + s * _CH1)
    nblocks = jnp.where(c == 0, _CH0 // _BCH, _CH1 // _BCH)

    def _issue_gather(i, q):
        pltpu.async_copy(wh_hbm.at[srcb.at[i]], rows.at[q], sems[q])

    def _process(i, p):
        for m in range(4):
            sv = srcb[i, pl.ds(m * 16, 16)]
            dv = dstb[i, pl.ds(m * 16, 16)]
            s1 = plsc.load_gather(ssrc_t, [sv])
            s2 = plsc.load_gather(sdst_t, [dv])
            z = s1 + s2
            e = jnp.where(z >= 0.0, z, 0.2 * z)
            ex = jnp.exp(e - jnp.maximum(s2 + gm, 0.0))
            exj[pl.ds(m * 16, 16)] = ex
        pltpu.sync_copy(exj, den_sh.at[dstb.at[i]], add=True)
        pltpu.make_async_copy(wh_hbm.at[srcb.at[i]], rows.at[p], sems[p]).wait()

        def _scale(g, carry):
            exv = exj[pl.ds(g * 16, 16)]
            for r16 in range(16):
                r = g * 16 + r16
                av = jnp.full((16,), exv[r16], jnp.float32)
                for q in range(8):
                    rows[p, r, pl.ds(q * 16, 16)] = (
                        rows[p, r, pl.ds(q * 16, 16)] * av)
            return carry

        lax.fori_loop(0, _CK // 16, _scale, 0)
        pltpu.sync_copy(rows.at[p], agg_sh.at[dstb.at[i]], add=True)

    def _block(b, carry):
        pltpu.sync_copy(src_hbm.at[pl.ds(start_ch + b * _BCH, _BCH)], srcb)
        pltpu.sync_copy(dst_hbm.at[pl.ds(start_ch + b * _BCH, _BCH)], dstb)
        _issue_gather(0, 0)

        def _pair(i2, carry2):
            i0 = 2 * i2
            _issue_gather(i0 + 1, 1)
            _process(i0, 0)
            _issue_gather(i0 + 2, 0)
            _process(i0 + 1, 1)
            return carry2

        lax.fori_loop(0, _BCH // 2 - 1, _pair, 0)
        _issue_gather(_BCH - 1, 1)
        _process(_BCH - 2, 0)
        _process(_BCH - 1, 1)
        return carry

    lax.fori_loop(0, nblocks, _block, 0)
    plsc.subcore_barrier()
    pltpu.sync_copy(agg_sh.at[pl.ds(base, _SLC)],
                    agg_out.at[c, pl.ds(base, _SLC)])
    pltpu.sync_copy(den_sh.at[pl.ds(base, _SLC)],
                    den_out.at[c, pl.ds(base, _SLC)])


def _sc_hop(ssrc, sdst, gmax16, srcp, dstp, wh):
    mesh = plsc.VectorSubcoreMesh(core_axis_name="c", subcore_axis_name="s")
    fn = pl.kernel(
        _sc_body,
        mesh=mesh,
        out_type=[jax.ShapeDtypeStruct((2, _NP, _H), _f32),
                  jax.ShapeDtypeStruct((2, _NP), _f32)],
        scratch_types=[
            pltpu.VMEM((_NP,), _f32),
            pltpu.VMEM((_NP,), _f32),
            pltpu.VMEM((_BCH, _CK), jnp.int32),
            pltpu.VMEM((_BCH, _CK), jnp.int32),
            pltpu.VMEM((_CK,), _f32),
            pltpu.VMEM((2, _CK, _H), _f32),
            pltpu.VMEM((16,), _f32),
            pltpu.VMEM_SHARED((_NP, _H), _f32),
            pltpu.VMEM_SHARED((_NP,), _f32),
            pltpu.SemaphoreType.DMA,
            pltpu.SemaphoreType.DMA,
        ],
        compiler_params=pltpu.CompilerParams(needs_layout_passes=False),
    )
    return fn(ssrc, sdst, gmax16, srcp, dstp, wh)


def kernel(x, edge_index, W_init, b_init, W_gat, a_src, a_dst,
           W_ih, W_hh, b_lstm, W_out, b_out):
    src = edge_index[0]
    dst = edge_index[1]
    srcp = jnp.concatenate(
        [src, jnp.zeros((_EP - _E,), jnp.int32)]).reshape(_TOTCH, _CK)
    dstp = jnp.concatenate(
        [dst, jnp.full((_EP - _E,), _NP - 1, jnp.int32)]).reshape(_TOTCH, _CK)
    xp = jnp.pad(x, ((0, _NP - _N), (0, 0)))

    bi = b_init.reshape(1, _H)
    bl = b_lstm.reshape(1, 4 * _H)
    Wo = jnp.pad(W_out, ((0, 0), (0, _H - _ODIM)))
    bo = jnp.pad(b_out, ((0, _H - _ODIM),)).reshape(1, _H)

    x0p, wh, ss, sd, gmax = _tc_pre(
        xp, W_init, bi, W_gat[0], a_src[0].reshape(_H, 1), a_dst[0].reshape(_H, 1))

    hs_list = []
    for hop in range(_HOPS):
        gmax16 = jnp.full((16,), gmax[0, 0], _f32)
        aggp, denp = _sc_hop(ss.reshape(_NP), sd.reshape(_NP), gmax16,
                             srcp, dstp, wh)
        d0 = denp[0].reshape(_NP, 1)
        d1 = denp[1].reshape(_NP, 1)
        if hop < _HOPS - 1:
            h, wh, ss, sd, gmax = _tc_mid(
                aggp, d0, d1, W_gat[hop + 1],
                a_src[hop + 1].reshape(_H, 1), a_dst[hop + 1].reshape(_H, 1))
            hs_list.append(h)
        else:
            (out,) = _tc_fin(aggp, d0, d1, x0p, hs_list[0], hs_list[1],
                             W_ih, W_hh, bl, Wo, bo)
    return out[:_N, :_ODIM]


# speedup vs baseline: 1.1721x; 1.0148x over previous
import jax
import jax.numpy as jnp
from jax import lax
from jax.experimental import pallas as pl
from jax.experimental.pallas import tpu as pltpu
from jax.experimental.pallas import tpu_sc as plsc

_N = 10000
_E = 320000
_D = 128
_H = 128
_ODIM = 121
_HOPS = 3

_NP = 10240
_NW = 32
_NT = 16
_CK = 64
_TOTCH = 5120
_BCH = 32
_CH0 = 256
_CH1 = 64
_EP = _TOTCH * _CK
_SLC = _NP // _NT
_BLK = 256
_GRID = _NP // _BLK


def _pre_body(x_ref, wi_ref, bi_ref, wg_ref, asrc_ref, adst_ref,
              x0_ref, wh_ref, ss_ref, sd_ref, gmax_ref):
    x0 = jnp.maximum(
        jnp.dot(x_ref[...], wi_ref[...], preferred_element_type=jnp.float32)
        + bi_ref[...], 0.0)
    x0_ref[...] = x0
    wh = jnp.dot(x0, wg_ref[...], preferred_element_type=jnp.float32)
    wh_ref[...] = wh
    ss = jnp.dot(wh, asrc_ref[...], preferred_element_type=jnp.float32)
    ss_ref[...] = ss
    sd_ref[...] = jnp.dot(wh, adst_ref[...], preferred_element_type=jnp.float32)
    m = jnp.full((1, 1), jnp.max(ss), jnp.float32)

    @pl.when(pl.program_id(0) == 0)
    def _():
        gmax_ref[...] = m

    @pl.when(pl.program_id(0) > 0)
    def _():
        gmax_ref[...] = jnp.maximum(gmax_ref[...], m)


def _mid_body(agg_ref, d0_ref, d1_ref, wg_ref, asrc_ref, adst_ref,
              h_ref, wh_ref, ss_ref, sd_ref, gmax_ref):
    invd = 1.0 / (d0_ref[...] + d1_ref[...] + 1e-16)
    h = jnp.tanh((agg_ref[0] + agg_ref[1]) * invd)
    h_ref[...] = h
    wh = jnp.dot(h, wg_ref[...], preferred_element_type=jnp.float32)
    wh_ref[...] = wh
    ss = jnp.dot(wh, asrc_ref[...], preferred_element_type=jnp.float32)
    ss_ref[...] = ss
    sd_ref[...] = jnp.dot(wh, adst_ref[...], preferred_element_type=jnp.float32)
    m = jnp.full((1, 1), jnp.max(ss), jnp.float32)

    @pl.when(pl.program_id(0) == 0)
    def _():
        gmax_ref[...] = m

    @pl.when(pl.program_id(0) > 0)
    def _():
        gmax_ref[...] = jnp.maximum(gmax_ref[...], m)


def _fin_body(agg_ref, d0_ref, d1_ref, x0_ref, h1_ref, h2_ref,
              wih_ref, whh_ref, bl_ref, wo_ref, bo_ref, out_ref):
    invd = 1.0 / (d0_ref[...] + d1_ref[...] + 1e-16)
    h3 = jnp.tanh((agg_ref[0] + agg_ref[1]) * invd)
    x0 = x0_ref[...]
    hs = jnp.zeros((_BLK, _H), jnp.float32)
    cs = jnp.zeros((_BLK, _H), jnp.float32)
    for ht in (h1_ref[...], h2_ref[...], h3):
        inp = ht + x0
        gates = (jnp.dot(inp, wih_ref[...], preferred_element_type=jnp.float32)
                 + jnp.dot(hs, whh_ref[...], preferred_element_type=jnp.float32)
                 + bl_ref[...])
        ig = jax.nn.sigmoid(gates[:, 0:_H])
        fg = jax.nn.sigmoid(gates[:, _H:2 * _H])
        gg = jnp.tanh(gates[:, 2 * _H:3 * _H])
        og = jax.nn.sigmoid(gates[:, 3 * _H:4 * _H])
        cs = fg * cs + ig * gg
        hs = og * jnp.tanh(cs)
    out_ref[...] = (jnp.dot(hs, wo_ref[...], preferred_element_type=jnp.float32)
                    + bo_ref[...])


_f32 = jnp.float32


def _vec_specs():
    full = lambda shape: pl.BlockSpec(shape, lambda i: (0,) * len(shape))
    row = pl.BlockSpec((_BLK, _D), lambda i: (i, 0))
    col = pl.BlockSpec((_BLK, 1), lambda i: (i, 0))
    return full, row, col


def _tc_pre(xp, W_init, b_init, Wg, asrc, adst):
    full, row, col = _vec_specs()
    return pl.pallas_call(
        _pre_body,
        grid=(_GRID,),
        in_specs=[row, full((_D, _H)), full((1, _H)), full((_H, _H)),
                  full((_H, 1)), full((_H, 1))],
        out_specs=[row, row, col, col, full((1, 1))],
        out_shape=[jax.ShapeDtypeStruct((_NP, _H), _f32),
                   jax.ShapeDtypeStruct((_NP, _H), _f32),
                   jax.ShapeDtypeStruct((_NP, 1), _f32),
                   jax.ShapeDtypeStruct((_NP, 1), _f32),
                   jax.ShapeDtypeStruct((1, 1), _f32)],
    )(xp, W_init, b_init, Wg, asrc, adst)


def _tc_mid(aggp, d0, d1, Wg, asrc, adst):
    full, row, col = _vec_specs()
    agg_spec = pl.BlockSpec((2, _BLK, _H), lambda i: (0, i, 0))
    return pl.pallas_call(
        _mid_body,
        grid=(_GRID,),
        in_specs=[agg_spec, col, col, full((_H, _H)),
                  full((_H, 1)), full((_H, 1))],
        out_specs=[row, row, col, col, full((1, 1))],
        out_shape=[jax.ShapeDtypeStruct((_NP, _H), _f32),
                   jax.ShapeDtypeStruct((_NP, _H), _f32),
                   jax.ShapeDtypeStruct((_NP, 1), _f32),
                   jax.ShapeDtypeStruct((_NP, 1), _f32),
                   jax.ShapeDtypeStruct((1, 1), _f32)],
    )(aggp, d0, d1, Wg, asrc, adst)


def _tc_fin(aggp, d0, d1, x0p, h1, h2, W_ih, W_hh, bl, Wo, bo):
    full, row, col = _vec_specs()
    agg_spec = pl.BlockSpec((2, _BLK, _H), lambda i: (0, i, 0))
    return pl.pallas_call(
        _fin_body,
        grid=(_GRID,),
        in_specs=[agg_spec, col, col, row, row, row,
                  full((_H, 4 * _H)), full((_H, 4 * _H)), full((1, 4 * _H)),
                  full((_H, _H)), full((1, _H))],
        out_specs=[row],
        out_shape=[jax.ShapeDtypeStruct((_NP, _H), _f32)],
    )(aggp, d0, d1, x0p, h1, h2, W_ih, W_hh, bl, Wo, bo)


def _sc_body(ssrc_hbm, sdst_hbm, gmax_hbm, src_hbm, dst_hbm, wh_hbm,
             agg_out, den_out,
             ssrc_t, sdst_t, srcb, dstb, exj, rows, gmaxv,
             agg_sh, den_sh, sem_g0, sem_g1):
    c = lax.axis_index("c")
    s = lax.axis_index("s")
    base = s * _SLC

    pltpu.sync_copy(ssrc_hbm, ssrc_t)
    pltpu.sync_copy(sdst_hbm, sdst_t)
    pltpu.sync_copy(gmax_hbm, gmaxv)

    zv = jnp.zeros((16,), jnp.float32)

    def _zero_rows(i, carry):
        for q in range(8):
            rows[0, i, pl.ds(q * 16, 16)] = zv
        return carry

    lax.fori_loop(0, _CK, _zero_rows, 0)
    for q in range(4):
        exj[pl.ds(q * 16, 16)] = zv

    for k in range(_SLC // _CK):
        pltpu.sync_copy(rows.at[0], agg_sh.at[pl.ds(base + k * _CK, _CK)])
        pltpu.sync_copy(exj, den_sh.at[pl.ds(base + k * _CK, _CK)])
    plsc.subcore_barrier()

    gm = gmaxv[...]
    sems = (sem_g0, sem_g1)

    start_ch = jnp.where(c == 0, s * _CH0, 16 * _CH0 + s * _CH1)
    nblocks = jnp.where(c == 0, _CH0 // _BCH, _CH1 // _BCH)

    def _issue_gather(i, q):
        pltpu.async_copy(wh_hbm.at[srcb.at[i]], rows.at[q], sems[q])

    def _process(i, p):
        for m in range(4):
            sv = srcb[i, pl.ds(m * 16, 16)]
            dv = dstb[i, pl.ds(m * 16, 16)]
            s1 = plsc.load_gather(ssrc_t, [sv])
            s2 = plsc.load_gather(sdst_t, [dv])
            z = s1 + s2
            e = jnp.where(z >= 0.0, z, 0.2 * z)
            ex = jnp.exp(e - jnp.maximum(s2 + gm, 0.0))
            exj[pl.ds(m * 16, 16)] = ex
        pltpu.sync_copy(exj, den_sh.at[dstb.at[i]], add=True)
        pltpu.make_async_copy(wh_hbm.at[srcb.at[i]], rows.at[p], sems[p]).wait()

        def _scale(g, carry):
            exv = exj[pl.ds(g * 16, 16)]
            for r16 in range(16):
                r = g * 16 + r16
                av = jnp.full((16,), exv[r16], jnp.float32)
                for q in range(8):
                    rows[p, r, pl.ds(q * 16, 16)] = (
                        rows[p, r, pl.ds(q * 16, 16)] * av)
            return carry

        lax.fori_loop(0, _CK // 16, _scale, 0)
        pltpu.sync_copy(rows.at[p], agg_sh.at[dstb.at[i]], add=True)

    def _block(b, carry):
        pltpu.sync_copy(src_hbm.at[pl.ds(start_ch + b * _BCH, _BCH)], srcb)
        pltpu.sync_copy(dst_hbm.at[pl.ds(start_ch + b * _BCH, _BCH)], dstb)
        _issue_gather(0, 0)

        def _pair(i2, carry2):
            i0 = 2 * i2
            _issue_gather(i0 + 1, 1)
            _process(i0, 0)
            _issue_gather(i0 + 2, 0)
            _process(i0 + 1, 1)
            return carry2

        lax.fori_loop(0, _BCH // 2 - 1, _pair, 0)
        _issue_gather(_BCH - 1, 1)
        _process(_BCH - 2, 0)
        _process(_BCH - 1, 1)
        return carry

    lax.fori_loop(0, nblocks, _block, 0)
    plsc.subcore_barrier()
    pltpu.sync_copy(agg_sh.at[pl.ds(base, _SLC)],
                    agg_out.at[c, pl.ds(base, _SLC)])
    pltpu.sync_copy(den_sh.at[pl.ds(base, _SLC)],
                    den_out.at[c, pl.ds(base, _SLC)])


def _sc_hop(ssrc, sdst, gmax16, srcp, dstp, wh):
    mesh = plsc.VectorSubcoreMesh(core_axis_name="c", subcore_axis_name="s")
    fn = pl.kernel(
        _sc_body,
        mesh=mesh,
        out_type=[jax.ShapeDtypeStruct((2, _NP, _H), _f32),
                  jax.ShapeDtypeStruct((2, _NP), _f32)],
        scratch_types=[
            pltpu.VMEM((_NP,), _f32),
            pltpu.VMEM((_NP,), _f32),
            pltpu.VMEM((_BCH, _CK), jnp.int32),
            pltpu.VMEM((_BCH, _CK), jnp.int32),
            pltpu.VMEM((_CK,), _f32),
            pltpu.VMEM((2, _CK, _H), _f32),
            pltpu.VMEM((16,), _f32),
            pltpu.VMEM_SHARED((_NP, _H), _f32),
            pltpu.VMEM_SHARED((_NP,), _f32),
            pltpu.SemaphoreType.DMA,
            pltpu.SemaphoreType.DMA,
        ],
        compiler_params=pltpu.CompilerParams(needs_layout_passes=False),
    )
    return fn(ssrc, sdst, gmax16, srcp, dstp, wh)


def kernel(x, edge_index, W_init, b_init, W_gat, a_src, a_dst,
           W_ih, W_hh, b_lstm, W_out, b_out):
    src = edge_index[0]
    dst = edge_index[1]
    srcp = jnp.concatenate(
        [src, jnp.zeros((_EP - _E,), jnp.int32)]).reshape(_TOTCH, _CK)
    dstp = jnp.concatenate(
        [dst, jnp.full((_EP - _E,), _NP - 1, jnp.int32)]).reshape(_TOTCH, _CK)
    xp = jnp.pad(x, ((0, _NP - _N), (0, 0)))

    bi = b_init.reshape(1, _H)
    bl = b_lstm.reshape(1, 4 * _H)
    Wo = jnp.pad(W_out, ((0, 0), (0, _H - _ODIM)))
    bo = jnp.pad(b_out, ((0, _H - _ODIM),)).reshape(1, _H)

    x0p, wh, ss, sd, gmax = _tc_pre(
        xp, W_init, bi, W_gat[0], a_src[0].reshape(_H, 1), a_dst[0].reshape(_H, 1))

    hs_list = []
    for hop in range(_HOPS):
        gmax16 = jnp.full((16,), gmax[0, 0], _f32)
        aggp, denp = _sc_hop(ss.reshape(_NP), sd.reshape(_NP), gmax16,
                             srcp, dstp, wh)
        d0 = denp[0].reshape(_NP, 1)
        d1 = denp[1].reshape(_NP, 1)
        if hop < _HOPS - 1:
            h, wh, ss, sd, gmax = _tc_mid(
                aggp, d0, d1, W_gat[hop + 1],
                a_src[hop + 1].reshape(_H, 1), a_dst[hop + 1].reshape(_H, 1))
            hs_list.append(h)
        else:
            (out,) = _tc_fin(aggp, d0, d1, x0p, hs_list[0], hs_list[1],
                             W_ih, W_hh, bl, Wo, bo)
    return out[:_N, :_ODIM]
